# baseline (speedup 1.0000x reference)
import jax
import jax.numpy as jnp
from jax.experimental import pallas as pl
from jax.experimental.pallas import tpu as pltpu

B, H, W = 1024, 30, 30
NV, DV = 10, 64
DP = 64
D = 128
BB = 8


def _embed_body(g_ref, ve_ref, pos_ref, out_ref):
    mask = jax.lax.broadcasted_iota(jnp.int32, (BB, W, D), 2) < DV
    rows = [jnp.concatenate([ve_ref[v], ve_ref[v]])[None, None, :]
            for v in range(NV)]
    for h in range(H):
        gb = jnp.broadcast_to(g_ref[:, h][..., None], (BB, W, D))
        gm = jnp.where(mask, gb, -1)
        pos_h = jnp.broadcast_to(pos_ref[h][None], (BB, W, DP))
        val0 = jnp.broadcast_to(ve_ref[0][None, None, :], (BB, W, DV))
        acc = jnp.concatenate([val0, pos_h], axis=-1)
        for v in range(1, NV):
            acc = jnp.where(gm == v, rows[v], acc)
        out_ref[:, h] = acc


def kernel(grid, value_embed, pos_encoding):
    g32 = grid.astype(jnp.int32)
    out = pl.pallas_call(
        _embed_body,
        grid=(B // BB,),
        in_specs=[
            pl.BlockSpec((BB, H, W), lambda i: (i, 0, 0)),
            pl.BlockSpec((NV, DV), lambda i: (0, 0)),
            pl.BlockSpec((H, W, DP), lambda i: (0, 0, 0)),
        ],
        out_specs=pl.BlockSpec((BB, H, W, D), lambda i: (i, 0, 0, 0)),
        out_shape=jax.ShapeDtypeStruct((B, H, W, D), jnp.float32),
    )(g32, value_embed, pos_encoding)
    return out


# speedup vs baseline: 1.3158x; 1.3158x over previous
import jax
import jax.numpy as jnp
from jax.experimental import pallas as pl
from jax.experimental.pallas import tpu as pltpu

B, H, W = 1024, 30, 30
NV, DV = 10, 64
DP = 64
D = 128
BB = 8


def _embed_body(g_ref, ve_ref, pos_ref, out_ref):
    x3 = jnp.broadcast_to(ve_ref[0:8][None], (BB, 8, DV))
    row8 = ve_ref[8][None, None, :]
    row9 = ve_ref[9][None, None, :]
    for h in range(H):
        idx3 = jnp.broadcast_to(g_ref[:, h][..., None], (BB, W, DV))
        val = jnp.take_along_axis(x3, idx3 & 7, axis=1,
                                  mode="promise_in_bounds")
        val = jnp.where(idx3 == 8, row8, val)
        val = jnp.where(idx3 == 9, row9, val)
        pos_h = jnp.broadcast_to(pos_ref[h][None], (BB, W, DP))
        out_ref[:, h] = jnp.concatenate([val, pos_h], axis=-1)


def kernel(grid, value_embed, pos_encoding):
    g32 = grid.astype(jnp.int32)
    out = pl.pallas_call(
        _embed_body,
        grid=(B // BB,),
        in_specs=[
            pl.BlockSpec((BB, H, W), lambda i: (i, 0, 0)),
            pl.BlockSpec((NV, DV), lambda i: (0, 0)),
            pl.BlockSpec((H, W, DP), lambda i: (0, 0, 0)),
        ],
        out_specs=pl.BlockSpec((BB, H, W, D), lambda i: (i, 0, 0, 0)),
        out_shape=jax.ShapeDtypeStruct((B, H, W, D), jnp.float32),
    )(g32, value_embed, pos_encoding)
    return out


# speedup vs baseline: 1.3624x; 1.0354x over previous
import jax
import jax.numpy as jnp
from jax.experimental import pallas as pl
from jax.experimental.pallas import tpu as pltpu

B, H, W = 1024, 30, 30
NV, DV = 10, 64
DP = 64
D = 128
BB = 32


def _embed_body(g_ref, ve_ref, pos_ref, out_ref):
    x3 = jnp.broadcast_to(ve_ref[0:8][None], (BB, 8, DV))
    row8 = ve_ref[8][None, None, :]
    row9 = ve_ref[9][None, None, :]
    for h in range(H):
        idx3 = jnp.broadcast_to(g_ref[:, h][..., None], (BB, W, DV))
        val = jnp.take_along_axis(x3, idx3 & 7, axis=1,
                                  mode="promise_in_bounds")
        val = jnp.where(idx3 == 8, row8, val)
        val = jnp.where(idx3 == 9, row9, val)
        pos_h = jnp.broadcast_to(pos_ref[h][None], (BB, W, DP))
        out_ref[:, h] = jnp.concatenate([val, pos_h], axis=-1)


def kernel(grid, value_embed, pos_encoding):
    g32 = grid.astype(jnp.int32)
    out = pl.pallas_call(
        _embed_body,
        grid=(B // BB,),
        in_specs=[
            pl.BlockSpec((BB, H, W), lambda i: (i, 0, 0)),
            pl.BlockSpec((NV, DV), lambda i: (0, 0)),
            pl.BlockSpec((H, W, DP), lambda i: (0, 0, 0)),
        ],
        out_specs=pl.BlockSpec((BB, H, W, D), lambda i: (i, 0, 0, 0)),
        out_shape=jax.ShapeDtypeStruct((B, H, W, D), jnp.float32),
    )(g32, value_embed, pos_encoding)
    return out
